# baseline (speedup 1.0000x reference)
import jax
import jax.numpy as jnp
from jax.experimental import pallas as pl

_N = 26


def _onehot_body(idxt_ref, out_ref):
    x = idxt_ref[...]
    k = pl.program_id(1)
    out_ref[...] = jnp.where(x[None, :, :] == k, 1, 0).astype(jnp.int32)


def kernel(idxs_vec):
    b, l = idxs_vec.shape
    idxt = idxs_vec.T
    h = 2
    c = b // h
    out3 = pl.pallas_call(
        _onehot_body,
        grid=(h, _N),
        in_specs=[pl.BlockSpec((l, c), lambda i, k: (0, i))],
        out_specs=pl.BlockSpec((1, l, c), lambda i, k: (k, 0, i)),
        out_shape=jax.ShapeDtypeStruct((_N, l, b), jnp.int32),
    )(idxt)
    return jnp.transpose(out3, (2, 1, 0))


# speedup vs baseline: 1.2666x; 1.2666x over previous
import jax
import jax.numpy as jnp
from jax.experimental import pallas as pl
from jax.experimental.pallas import tpu as pltpu

_N = 26
_NBUF = 8


def _onehot_body(idxt_ref, out_ref, scratch, sems):
    x = idxt_ref[...]
    for k in range(_N):
        buf = k % _NBUF
        if k >= _NBUF:
            pltpu.make_async_copy(
                scratch.at[buf], out_ref.at[k - _NBUF], sems.at[buf]
            ).wait()
        scratch[buf, :, :] = jnp.where(x == k, 1, 0).astype(jnp.int32)
        pltpu.make_async_copy(scratch.at[buf], out_ref.at[k], sems.at[buf]).start()
    for k in range(_N - _NBUF, _N):
        buf = k % _NBUF
        pltpu.make_async_copy(scratch.at[buf], out_ref.at[k], sems.at[buf]).wait()


def kernel(idxs_vec):
    b, l = idxs_vec.shape
    idxt = idxs_vec.T
    out3 = pl.pallas_call(
        _onehot_body,
        in_specs=[pl.BlockSpec((l, b), lambda: (0, 0))],
        out_specs=pl.BlockSpec(memory_space=pl.ANY),
        out_shape=jax.ShapeDtypeStruct((_N, l, b), jnp.int32),
        scratch_shapes=[
            pltpu.VMEM((_NBUF, l, b), jnp.int32),
            pltpu.SemaphoreType.DMA((_NBUF,)),
        ],
    )(idxt)
    return jnp.transpose(out3, (2, 1, 0))


# speedup vs baseline: 1.3162x; 1.0392x over previous
import jax
import jax.numpy as jnp
from jax.experimental import pallas as pl

_N = 26
_KB = 2


def _onehot_body(idxt_ref, out_ref):
    x = idxt_ref[...]
    l, b = x.shape
    k0 = pl.program_id(0) * _KB
    k = k0 + jax.lax.broadcasted_iota(jnp.int32, (_KB, l, b), 0)
    out_ref[...] = jnp.where(x[None, :, :] == k, 1, 0).astype(jnp.int32)


def kernel(idxs_vec):
    b, l = idxs_vec.shape
    idxt = idxs_vec.T
    out3 = pl.pallas_call(
        _onehot_body,
        grid=(_N // _KB,),
        in_specs=[pl.BlockSpec((l, b), lambda k: (0, 0))],
        out_specs=pl.BlockSpec((_KB, l, b), lambda k: (k, 0, 0)),
        out_shape=jax.ShapeDtypeStruct((_N, l, b), jnp.int32),
    )(idxt)
    return jnp.transpose(out3, (2, 1, 0))
